# baseline (speedup 1.0000x reference)
import jax
import jax.numpy as jnp
from jax import lax
from jax.experimental import pallas as pl
from jax.experimental.pallas import tpu as pltpu
from jax.experimental.pallas import tpu_sc as plsc

_NC = 2
_NS = 16
_NW = _NC * _NS


def kernel(rank_idxs, suit_idxs, rank_table, suit_table):
    B, H = rank_idxs.shape
    V_rank, D = rank_table.shape

    table = jnp.concatenate([rank_table, suit_table], axis=0)

    idx = jnp.stack(
        [rank_idxs.astype(jnp.int32), suit_idxs.astype(jnp.int32) + V_rank],
        axis=-1,
    ).reshape(B * H * 2)

    N = B * H * 2
    C = 32
    assert N % (_NW * C) == 0
    per_w = N // _NW
    steps = per_w // C

    mesh = plsc.VectorSubcoreMesh(core_axis_name="c", subcore_axis_name="s")

    @jax.jit
    def run(table, idx):
        @pl.kernel(
            out_type=jax.ShapeDtypeStruct((N, D), jnp.float32),
            mesh=mesh,
            scratch_types=[
                pltpu.VMEM((C,), jnp.int32),
                pltpu.VMEM((C, D), jnp.float32),
                pltpu.SemaphoreType.DMA,
            ],
        )
        def k(table_hbm, idx_hbm, out_hbm, idx_v, rows_v, sem):
            wid = lax.axis_index("s") * _NC + lax.axis_index("c")
            base = wid * per_w

            @pl.loop(0, steps)
            def _(it):
                off = base + it * C
                pltpu.sync_copy(idx_hbm.at[pl.ds(off, C)], idx_v)
                pltpu.async_copy(table_hbm.at[idx_v], rows_v, sem).wait()
                pltpu.sync_copy(rows_v, out_hbm.at[pl.ds(off, C)])

        return k(table, idx)

    out = run(table, idx)
    return out.reshape(B, H, 2 * D)
